# Initial kernel scaffold; baseline (speedup 1.0000x reference)
import jax
import jax.numpy as jnp
from jax.experimental import pallas as pl


def kernel(boxes, scores):
    raise NotImplementedError("write your pallas kernel here")


# speedup vs baseline: 116.6372x; 116.6372x over previous
import jax
import jax.numpy as jnp
from jax.experimental import pallas as pl
from jax.experimental.pallas import tpu as pltpu

_N = 5000
_NPAD = 5120
_B = 512
_NB = _NPAD // _B
_T = 0.5


def _nms_body(bc_ref, br_ref, ac_ref, ar_ref, s_ref, keep_out, ks_out, keep_ref):
    lane = jax.lax.broadcasted_iota(jnp.int32, (8, _NPAD), 1)
    keep_ref[...] = jnp.where(lane < _N, 1.0, 0.0).astype(jnp.float32)

    row_i = jax.lax.broadcasted_iota(jnp.int32, (_B, _B), 0)
    col_i = jax.lax.broadcasted_iota(jnp.int32, (_B, _B), 1)
    upper = col_i > row_i

    def iou_block(i, j):
        r0 = i * _B
        c0 = j * _B
        x1r = bc_ref[r0:r0 + _B, 0:1]
        y1r = bc_ref[r0:r0 + _B, 1:2]
        x2r = bc_ref[r0:r0 + _B, 2:3]
        y2r = bc_ref[r0:r0 + _B, 3:4]
        ar = ac_ref[r0:r0 + _B, 0:1]
        x1c = br_ref[0:1, c0:c0 + _B]
        y1c = br_ref[1:2, c0:c0 + _B]
        x2c = br_ref[2:3, c0:c0 + _B]
        y2c = br_ref[3:4, c0:c0 + _B]
        ac = ar_ref[0:1, c0:c0 + _B]
        xx1 = jnp.maximum(x1r, x1c)
        yy1 = jnp.maximum(y1r, y1c)
        xx2 = jnp.minimum(x2r, x2c)
        yy2 = jnp.minimum(y2r, y2c)
        w = jnp.maximum(xx2 - xx1, 0.0)
        h = jnp.maximum(yy2 - yy1, 0.0)
        inter = w * h
        iou = inter / (ar + ac - inter + 1e-9)
        return iou > _T

    for i in range(_NB):
        r0 = i * _B
        d = jnp.where(iou_block(i, i) & upper, 1.0, 0.0).astype(jnp.float32)
        init = keep_ref[:, r0:r0 + _B]

        def cond(c):
            return c[1]

        def body(c):
            k, _ = c
            sup = jnp.dot(k, d, preferred_element_type=jnp.float32)
            kn = jnp.where(sup > 0.5, 0.0, init)
            return kn, jnp.any(kn != k)

        k, _ = jax.lax.while_loop(cond, body, (init, jnp.bool_(True)))
        keep_ref[:, r0:r0 + _B] = k

        for j in range(i + 1, _NB):
            c0 = j * _B
            s_ij = jnp.where(iou_block(i, j), 1.0, 0.0).astype(jnp.float32)
            sup = jnp.dot(k, s_ij, preferred_element_type=jnp.float32)
            keep_ref[:, c0:c0 + _B] = jnp.where(
                sup > 0.5, 0.0, keep_ref[:, c0:c0 + _B])

    keep_out[...] = keep_ref[0:1, :]
    ks_out[...] = s_ref[...] * keep_ref[0:1, :]


def _nms_pallas(bc, br, ac, ar, s):
    return pl.pallas_call(
        _nms_body,
        out_shape=(
            jax.ShapeDtypeStruct((1, _NPAD), jnp.float32),
            jax.ShapeDtypeStruct((1, _NPAD), jnp.float32),
        ),
        scratch_shapes=[pltpu.VMEM((8, _NPAD), jnp.float32)],
    )(bc, br, ac, ar, s)


def kernel(boxes, scores):
    order = jnp.argsort(-scores)
    b = boxes[order]
    s = scores[order]
    areas = (b[:, 2] - b[:, 0]) * (b[:, 3] - b[:, 1])

    pad = _NPAD - _N
    bc = jnp.pad(b, ((0, pad), (0, 0)))
    br = jnp.pad(b.T, ((0, 0), (0, pad)))
    ac = jnp.pad(areas[:, None], ((0, pad), (0, 0)))
    ar = jnp.pad(areas[None, :], ((0, 0), (0, pad)))
    sp = jnp.pad(s[None, :], ((0, 0), (0, pad)))

    keep_f, ks = _nms_pallas(bc, br, ac, ar, sp)
    keep = keep_f[0, :_N] > 0.5
    kept_scores = ks[0, :_N]
    return kept_scores, keep, order


# speedup vs baseline: 134.6697x; 1.1546x over previous
import functools

import jax
import jax.numpy as jnp
from jax import lax
from jax.experimental import pallas as pl
from jax.experimental.pallas import tpu as pltpu
from jax.experimental.pallas import tpu_sc as plsc

_N = 5000
_NPAD = 5120
_B = 512
_NB = _NPAD // _B
_T = 0.5

_D = 128
_NW = 32
_BPW = _NPAD // _NW


@functools.cache
def _make_sc_gather():
    mesh = plsc.VectorSubcoreMesh(core_axis_name="c", subcore_axis_name="s")

    @functools.partial(
        pl.kernel,
        mesh=mesh,
        out_type=jax.ShapeDtypeStruct((_NPAD, _D), jnp.float32),
        scratch_types=[
            pltpu.VMEM((_BPW,), jnp.int32),
            pltpu.VMEM((_BPW, _D), jnp.float32),
            pltpu.SemaphoreType.DMA,
        ],
    )
    def gather_rows(table_hbm, idx_hbm, out_hbm, idx_v, rows_v, sem):
        wid = lax.axis_index("s") * 2 + lax.axis_index("c")
        base = wid * _BPW
        pltpu.sync_copy(idx_hbm.at[pl.ds(base, _BPW)], idx_v)
        pltpu.async_copy(table_hbm.at[idx_v], rows_v, sem).wait()
        pltpu.sync_copy(rows_v, out_hbm.at[pl.ds(base, _BPW)])

    return gather_rows


def _nms_body(bt_ref, btt_ref, s_ref, keep_out, ks_out, keep_ref):
    lane = jax.lax.broadcasted_iota(jnp.int32, (8, _NPAD), 1)
    keep_ref[...] = jnp.where(lane < _N, 1.0, 0.0).astype(jnp.float32)

    row_i = jax.lax.broadcasted_iota(jnp.int32, (_B, _B), 0)
    col_i = jax.lax.broadcasted_iota(jnp.int32, (_B, _B), 1)
    upper = col_i > row_i

    area_col = (bt_ref[:, 2:3] - bt_ref[:, 0:1]) * (bt_ref[:, 3:4] - bt_ref[:, 1:2])
    area_row = (btt_ref[2:3, :] - btt_ref[0:1, :]) * (btt_ref[3:4, :] - btt_ref[1:2, :])

    def iou_block(i, j):
        r0 = i * _B
        c0 = j * _B
        x1r = bt_ref[r0:r0 + _B, 0:1]
        y1r = bt_ref[r0:r0 + _B, 1:2]
        x2r = bt_ref[r0:r0 + _B, 2:3]
        y2r = bt_ref[r0:r0 + _B, 3:4]
        ar = area_col[r0:r0 + _B, :]
        x1c = btt_ref[0:1, c0:c0 + _B]
        y1c = btt_ref[1:2, c0:c0 + _B]
        x2c = btt_ref[2:3, c0:c0 + _B]
        y2c = btt_ref[3:4, c0:c0 + _B]
        ac = area_row[:, c0:c0 + _B]
        xx1 = jnp.maximum(x1r, x1c)
        yy1 = jnp.maximum(y1r, y1c)
        xx2 = jnp.minimum(x2r, x2c)
        yy2 = jnp.minimum(y2r, y2c)
        w = jnp.maximum(xx2 - xx1, 0.0)
        h = jnp.maximum(yy2 - yy1, 0.0)
        inter = w * h
        iou = inter / (ar + ac - inter + 1e-9)
        return iou > _T

    for i in range(_NB):
        r0 = i * _B
        d = jnp.where(iou_block(i, i) & upper, 1.0, 0.0).astype(jnp.float32)
        init = keep_ref[:, r0:r0 + _B]

        def cond(c):
            return c[1]

        def body(c):
            k, _ = c
            sup = jnp.dot(k, d, preferred_element_type=jnp.float32)
            kn = jnp.where(sup > 0.5, 0.0, init)
            return kn, jnp.any(kn != k)

        k, _ = jax.lax.while_loop(cond, body, (init, jnp.bool_(True)))
        keep_ref[:, r0:r0 + _B] = k

        for j in range(i + 1, _NB):
            c0 = j * _B
            s_ij = jnp.where(iou_block(i, j), 1.0, 0.0).astype(jnp.float32)
            sup = jnp.dot(k, s_ij, preferred_element_type=jnp.float32)
            keep_ref[:, c0:c0 + _B] = jnp.where(
                sup > 0.5, 0.0, keep_ref[:, c0:c0 + _B])

    keep_out[...] = keep_ref[0:1, :]
    ks_out[...] = s_ref[...] * keep_ref[0:1, :]


def _nms_pallas(bt, btt, s):
    return pl.pallas_call(
        _nms_body,
        out_shape=(
            jax.ShapeDtypeStruct((1, _NPAD), jnp.float32),
            jax.ShapeDtypeStruct((1, _NPAD), jnp.float32),
        ),
        scratch_shapes=[pltpu.VMEM((8, _NPAD), jnp.float32)],
    )(bt, btt, s)


def kernel(boxes, scores):
    neg_sorted, order = lax.sort_key_val(-scores, jnp.arange(_N, dtype=jnp.int32))
    s = -neg_sorted

    table = jnp.pad(boxes, ((0, 0), (0, _D - 4)))
    idx = jnp.pad(order, (0, _NPAD - _N))
    bt = _make_sc_gather()(table, idx)[:, :4]
    btt = bt.T

    sp = jnp.pad(s, (0, _NPAD - _N))[None, :]

    keep_f, ks = _nms_pallas(bt, btt, sp)
    keep = keep_f[0, :_N] > 0.5
    kept_scores = ks[0, :_N]
    return kept_scores, keep, order


# speedup vs baseline: 136.5796x; 1.0142x over previous
import functools

import jax
import jax.numpy as jnp
from jax import lax
from jax.experimental import pallas as pl
from jax.experimental.pallas import tpu as pltpu
from jax.experimental.pallas import tpu_sc as plsc

_N = 5000
_NPAD = 5120
_B = 512
_NB = _NPAD // _B
_T = 0.5

_D = 128
_NW = 32
_BPW = _NPAD // _NW


@functools.cache
def _make_sc_gather():
    mesh = plsc.VectorSubcoreMesh(core_axis_name="c", subcore_axis_name="s")

    @functools.partial(
        pl.kernel,
        mesh=mesh,
        out_type=jax.ShapeDtypeStruct((_NPAD, _D), jnp.float32),
        scratch_types=[
            pltpu.VMEM((_BPW,), jnp.int32),
            pltpu.VMEM((_BPW, _D), jnp.float32),
            pltpu.SemaphoreType.DMA,
        ],
    )
    def gather_rows(table_hbm, idx_hbm, out_hbm, idx_v, rows_v, sem):
        wid = lax.axis_index("s") * 2 + lax.axis_index("c")
        base = wid * _BPW
        pltpu.sync_copy(idx_hbm.at[pl.ds(base, _BPW)], idx_v)
        pltpu.async_copy(table_hbm.at[idx_v], rows_v, sem).wait()
        pltpu.sync_copy(rows_v, out_hbm.at[pl.ds(base, _BPW)])

    return gather_rows


def _nms_body(bt_ref, btt_ref, s_ref, keep_out, ks_out, keep_ref):
    lane = jax.lax.broadcasted_iota(jnp.int32, (8, _NPAD), 1)
    keep_ref[...] = jnp.where(lane < _N, 1.0, 0.0).astype(jnp.float32)

    row_i = jax.lax.broadcasted_iota(jnp.int32, (_B, _B), 0)
    col_i = jax.lax.broadcasted_iota(jnp.int32, (_B, _B), 1)
    upper = col_i > row_i

    area_col = (bt_ref[:, 2:3] - bt_ref[:, 0:1]) * (bt_ref[:, 3:4] - bt_ref[:, 1:2])
    area_row = (btt_ref[2:3, :] - btt_ref[0:1, :]) * (btt_ref[3:4, :] - btt_ref[1:2, :])

    def iou_block(i, j):
        r0 = i * _B
        c0 = j * _B
        x1r = bt_ref[r0:r0 + _B, 0:1]
        y1r = bt_ref[r0:r0 + _B, 1:2]
        x2r = bt_ref[r0:r0 + _B, 2:3]
        y2r = bt_ref[r0:r0 + _B, 3:4]
        ar = area_col[r0:r0 + _B, :]
        x1c = btt_ref[0:1, c0:c0 + _B]
        y1c = btt_ref[1:2, c0:c0 + _B]
        x2c = btt_ref[2:3, c0:c0 + _B]
        y2c = btt_ref[3:4, c0:c0 + _B]
        ac = area_row[:, c0:c0 + _B]
        xx1 = jnp.maximum(x1r, x1c)
        yy1 = jnp.maximum(y1r, y1c)
        xx2 = jnp.minimum(x2r, x2c)
        yy2 = jnp.minimum(y2r, y2c)
        w = jnp.maximum(xx2 - xx1, 0.0)
        h = jnp.maximum(yy2 - yy1, 0.0)
        inter = w * h
        iou = inter / (ar + ac - inter + 1e-9)
        return iou > _T

    for i in range(_NB):
        r0 = i * _B
        d = jnp.where(iou_block(i, i) & upper, 1.0, 0.0).astype(jnp.float32)
        init = keep_ref[:, r0:r0 + _B]

        def cond(c):
            return c[1]

        def body(c):
            k, _ = c
            sup = jnp.dot(k, d, preferred_element_type=jnp.float32)
            k1 = jnp.where(sup > 0.5, 0.0, init)
            sup2 = jnp.dot(k1, d, preferred_element_type=jnp.float32)
            k2 = jnp.where(sup2 > 0.5, 0.0, init)
            return k2, jnp.any(k2 != k1)

        k, _ = jax.lax.while_loop(cond, body, (init, jnp.bool_(True)))
        keep_ref[:, r0:r0 + _B] = k

        for j in range(i + 1, _NB):
            c0 = j * _B
            s_ij = jnp.where(iou_block(i, j), 1.0, 0.0).astype(jnp.float32)
            sup = jnp.dot(k, s_ij, preferred_element_type=jnp.float32)
            keep_ref[:, c0:c0 + _B] = jnp.where(
                sup > 0.5, 0.0, keep_ref[:, c0:c0 + _B])

    keep_out[...] = keep_ref[0:1, :]
    ks_out[...] = s_ref[...] * keep_ref[0:1, :]


def _nms_pallas(bt, btt, s):
    return pl.pallas_call(
        _nms_body,
        out_shape=(
            jax.ShapeDtypeStruct((1, _NPAD), jnp.float32),
            jax.ShapeDtypeStruct((1, _NPAD), jnp.float32),
        ),
        scratch_shapes=[pltpu.VMEM((8, _NPAD), jnp.float32)],
    )(bt, btt, s)


def kernel(boxes, scores):
    neg_sorted, order = lax.sort_key_val(-scores, jnp.arange(_N, dtype=jnp.int32))
    s = -neg_sorted

    table = jnp.pad(boxes, ((0, 0), (0, _D - 4)))
    idx = jnp.pad(order, (0, _NPAD - _N))
    bt = _make_sc_gather()(table, idx)[:, :4]
    btt = bt.T

    sp = jnp.pad(s, (0, _NPAD - _N))[None, :]

    keep_f, ks = _nms_pallas(bt, btt, sp)
    keep = keep_f[0, :_N] > 0.5
    kept_scores = ks[0, :_N]
    return kept_scores, keep, order


# speedup vs baseline: 146.1840x; 1.0703x over previous
import functools

import jax
import jax.numpy as jnp
from jax import lax
from jax.experimental import pallas as pl
from jax.experimental.pallas import tpu as pltpu
from jax.experimental.pallas import tpu_sc as plsc

_N = 5000
_NPAD = 5120
_B = 1024
_NB = _NPAD // _B
_T = 0.5

_D = 128
_NW = 32
_BPW = _NPAD // _NW


@functools.cache
def _make_sc_gather():
    mesh = plsc.VectorSubcoreMesh(core_axis_name="c", subcore_axis_name="s")

    @functools.partial(
        pl.kernel,
        mesh=mesh,
        out_type=jax.ShapeDtypeStruct((_NPAD, _D), jnp.float32),
        scratch_types=[
            pltpu.VMEM((_BPW,), jnp.int32),
            pltpu.VMEM((_BPW, _D), jnp.float32),
            pltpu.SemaphoreType.DMA,
        ],
    )
    def gather_rows(table_hbm, idx_hbm, out_hbm, idx_v, rows_v, sem):
        wid = lax.axis_index("s") * 2 + lax.axis_index("c")
        base = wid * _BPW
        pltpu.sync_copy(idx_hbm.at[pl.ds(base, _BPW)], idx_v)
        pltpu.async_copy(table_hbm.at[idx_v], rows_v, sem).wait()
        pltpu.sync_copy(rows_v, out_hbm.at[pl.ds(base, _BPW)])

    return gather_rows


def _nms_body(bt_ref, btt_ref, s_ref, keep_out, ks_out, keep_ref):
    lane = jax.lax.broadcasted_iota(jnp.int32, (8, _NPAD), 1)
    keep_ref[...] = jnp.where(lane < _N, 1.0, 0.0).astype(jnp.float32)

    row_i = jax.lax.broadcasted_iota(jnp.int32, (_B, _B), 0)
    col_i = jax.lax.broadcasted_iota(jnp.int32, (_B, _B), 1)
    upper = col_i > row_i

    area_col = (bt_ref[:, 2:3] - bt_ref[:, 0:1]) * (bt_ref[:, 3:4] - bt_ref[:, 1:2])
    area_row = (btt_ref[2:3, :] - btt_ref[0:1, :]) * (btt_ref[3:4, :] - btt_ref[1:2, :])

    def row_tiles(i):
        r0 = i * _B
        return tuple(
            jnp.broadcast_to(v, (_B, _B))
            for v in (bt_ref[r0:r0 + _B, 0:1], bt_ref[r0:r0 + _B, 1:2],
                      bt_ref[r0:r0 + _B, 2:3], bt_ref[r0:r0 + _B, 3:4],
                      area_col[r0:r0 + _B, :]))

    def iou_block(rows, j):
        c0 = j * _B
        x1r, y1r, x2r, y2r, ar = rows
        x1c = btt_ref[0:1, c0:c0 + _B]
        y1c = btt_ref[1:2, c0:c0 + _B]
        x2c = btt_ref[2:3, c0:c0 + _B]
        y2c = btt_ref[3:4, c0:c0 + _B]
        ac = area_row[:, c0:c0 + _B]
        xx1 = jnp.maximum(x1r, x1c)
        yy1 = jnp.maximum(y1r, y1c)
        xx2 = jnp.minimum(x2r, x2c)
        yy2 = jnp.minimum(y2r, y2c)
        w = jnp.maximum(xx2 - xx1, 0.0)
        h = jnp.maximum(yy2 - yy1, 0.0)
        inter = w * h
        iou = inter / (ar + ac - inter + 1e-9)
        return iou > _T

    for i in range(_NB):
        r0 = i * _B
        rows = row_tiles(i)
        d = jnp.where(iou_block(rows, i) & upper, 1.0, 0.0).astype(jnp.float32)
        init = keep_ref[:, r0:r0 + _B]

        def cond(c):
            return c[1]

        def body(c):
            k, _ = c
            sup = jnp.dot(k, d, preferred_element_type=jnp.float32)
            k1 = jnp.where(sup > 0.5, 0.0, init)
            sup2 = jnp.dot(k1, d, preferred_element_type=jnp.float32)
            k2 = jnp.where(sup2 > 0.5, 0.0, init)
            return k2, jnp.any(k2 != k1)

        k, _ = jax.lax.while_loop(cond, body, (init, jnp.bool_(True)))
        keep_ref[:, r0:r0 + _B] = k

        for j in range(i + 1, _NB):
            c0 = j * _B
            s_ij = jnp.where(iou_block(rows, j), 1.0, 0.0).astype(jnp.float32)
            sup = jnp.dot(k, s_ij, preferred_element_type=jnp.float32)
            keep_ref[:, c0:c0 + _B] = jnp.where(
                sup > 0.5, 0.0, keep_ref[:, c0:c0 + _B])

    keep_out[...] = keep_ref[0:1, :]
    ks_out[...] = s_ref[...] * keep_ref[0:1, :]


def _nms_pallas(bt, btt, s):
    return pl.pallas_call(
        _nms_body,
        out_shape=(
            jax.ShapeDtypeStruct((1, _NPAD), jnp.float32),
            jax.ShapeDtypeStruct((1, _NPAD), jnp.float32),
        ),
        scratch_shapes=[pltpu.VMEM((8, _NPAD), jnp.float32)],
    )(bt, btt, s)


def kernel(boxes, scores):
    neg_sorted, order = lax.sort_key_val(-scores, jnp.arange(_N, dtype=jnp.int32))
    s = -neg_sorted

    table = jnp.pad(boxes, ((0, 0), (0, _D - 4)))
    idx = jnp.pad(order, (0, _NPAD - _N))
    bt = _make_sc_gather()(table, idx)[:, :4]
    btt = bt.T

    sp = jnp.pad(s, (0, _NPAD - _N))[None, :]

    keep_f, ks = _nms_pallas(bt, btt, sp)
    keep = keep_f[0, :_N] > 0.5
    kept_scores = ks[0, :_N]
    return kept_scores, keep, order


# speedup vs baseline: 151.5052x; 1.0364x over previous
import functools

import jax
import jax.numpy as jnp
from jax import lax
from jax.experimental import pallas as pl
from jax.experimental.pallas import tpu as pltpu
from jax.experimental.pallas import tpu_sc as plsc

_N = 5000
_NPAD = 5120
_B = 512
_NB = _NPAD // _B
_T = 0.5

_D = 128
_NW = 32
_BPW = _NPAD // _NW


@functools.cache
def _make_sc_gather():
    mesh = plsc.VectorSubcoreMesh(core_axis_name="c", subcore_axis_name="s")

    @functools.partial(
        pl.kernel,
        mesh=mesh,
        out_type=jax.ShapeDtypeStruct((_NPAD, _D), jnp.float32),
        scratch_types=[
            pltpu.VMEM((_BPW,), jnp.int32),
            pltpu.VMEM((_BPW, _D), jnp.float32),
            pltpu.SemaphoreType.DMA,
        ],
    )
    def gather_rows(table_hbm, idx_hbm, out_hbm, idx_v, rows_v, sem):
        wid = lax.axis_index("s") * 2 + lax.axis_index("c")
        base = wid * _BPW
        pltpu.sync_copy(idx_hbm.at[pl.ds(base, _BPW)], idx_v)
        pltpu.async_copy(table_hbm.at[idx_v], rows_v, sem).wait()
        pltpu.sync_copy(rows_v, out_hbm.at[pl.ds(base, _BPW)])

    return gather_rows


def _nms_body(bt_ref, btt_ref, s_ref, keep_out, ks_out, keep_ref):
    lane = jax.lax.broadcasted_iota(jnp.int32, (8, _NPAD), 1)
    keep_ref[...] = jnp.where(lane < _N, 1.0, 0.0).astype(jnp.float32)

    row_i = jax.lax.broadcasted_iota(jnp.int32, (_B, _B), 0)
    col_i = jax.lax.broadcasted_iota(jnp.int32, (_B, _B), 1)
    upper = col_i > row_i

    area_col = (bt_ref[:, 2:3] - bt_ref[:, 0:1]) * (bt_ref[:, 3:4] - bt_ref[:, 1:2])
    area_row = (btt_ref[2:3, :] - btt_ref[0:1, :]) * (btt_ref[3:4, :] - btt_ref[1:2, :])

    def row_tiles(i):
        r0 = i * _B
        return tuple(
            jnp.broadcast_to(v, (_B, _B))
            for v in (bt_ref[r0:r0 + _B, 0:1], bt_ref[r0:r0 + _B, 1:2],
                      bt_ref[r0:r0 + _B, 2:3], bt_ref[r0:r0 + _B, 3:4],
                      area_col[r0:r0 + _B, :]))

    def iou_block(rows, j):
        c0 = j * _B
        x1r, y1r, x2r, y2r, ar = rows
        x1c = btt_ref[0:1, c0:c0 + _B]
        y1c = btt_ref[1:2, c0:c0 + _B]
        x2c = btt_ref[2:3, c0:c0 + _B]
        y2c = btt_ref[3:4, c0:c0 + _B]
        ac = area_row[:, c0:c0 + _B]
        xx1 = jnp.maximum(x1r, x1c)
        yy1 = jnp.maximum(y1r, y1c)
        xx2 = jnp.minimum(x2r, x2c)
        yy2 = jnp.minimum(y2r, y2c)
        w = jnp.maximum(xx2 - xx1, 0.0)
        h = jnp.maximum(yy2 - yy1, 0.0)
        inter = w * h
        iou = inter / (ar + ac - inter + 1e-9)
        return iou > _T

    for i in range(_NB):
        r0 = i * _B
        rows = row_tiles(i)
        d = jnp.where(iou_block(rows, i) & upper, 1.0, 0.0).astype(jnp.float32)
        init = keep_ref[:, r0:r0 + _B]

        def cond(c):
            return c[1]

        def body(c):
            k, _ = c
            sup = jnp.dot(k, d, preferred_element_type=jnp.float32)
            k1 = jnp.where(sup > 0.5, 0.0, init)
            sup2 = jnp.dot(k1, d, preferred_element_type=jnp.float32)
            k2 = jnp.where(sup2 > 0.5, 0.0, init)
            return k2, jnp.any(k2 != k1)

        k, _ = jax.lax.while_loop(cond, body, (init, jnp.bool_(True)))
        keep_ref[:, r0:r0 + _B] = k

        for j in range(i + 1, _NB):
            c0 = j * _B
            s_ij = jnp.where(iou_block(rows, j), 1.0, 0.0).astype(jnp.float32)
            sup = jnp.dot(k, s_ij, preferred_element_type=jnp.float32)
            keep_ref[:, c0:c0 + _B] = jnp.where(
                sup > 0.5, 0.0, keep_ref[:, c0:c0 + _B])

    keep_out[...] = keep_ref[0:1, :]
    ks_out[...] = s_ref[...] * keep_ref[0:1, :]


def _nms_pallas(bt, btt, s):
    return pl.pallas_call(
        _nms_body,
        out_shape=(
            jax.ShapeDtypeStruct((1, _NPAD), jnp.float32),
            jax.ShapeDtypeStruct((1, _NPAD), jnp.float32),
        ),
        scratch_shapes=[pltpu.VMEM((8, _NPAD), jnp.float32)],
    )(bt, btt, s)


def kernel(boxes, scores):
    neg_sorted, order = lax.sort_key_val(-scores, jnp.arange(_N, dtype=jnp.int32))
    s = -neg_sorted

    table = jnp.pad(boxes, ((0, 0), (0, _D - 4)))
    idx = jnp.pad(order, (0, _NPAD - _N))
    bt = _make_sc_gather()(table, idx)[:, :4]
    btt = bt.T

    sp = jnp.pad(s, (0, _NPAD - _N))[None, :]

    keep_f, ks = _nms_pallas(bt, btt, sp)
    keep = keep_f[0, :_N] > 0.5
    kept_scores = ks[0, :_N]
    return kept_scores, keep, order
